# baseline (speedup 1.0000x reference)
import jax
import jax.numpy as jnp
from jax.experimental import pallas as pl
from jax.experimental.pallas import tpu as pltpu

_NE = 1024
_NL = 8
_NF = 32
_E = 128


def _off(level):
    return 2 ** level - 1


def _bitrev(j, bits):
    r = 0
    for _ in range(bits):
        r = (r << 1) | (j & 1)
        j >>= 1
    return r


def _leaky(x):
    return jnp.maximum(x, 0.1 * x)


def _mlp(params, x):
    n = len(params)
    for i, (Wt, b) in enumerate(params):
        x = jnp.dot(Wt, x, preferred_element_type=jnp.float32) + b
        if i < n - 1:
            x = _leaky(x)
    return x


def _fold_sum(x, steps):
    for _ in range(steps):
        h = x.shape[1] // 2
        x = x[:, :h] + x[:, h:]
    return x


def _fold_max(x, steps):
    for _ in range(steps):
        h = x.shape[1] // 2
        x = jnp.maximum(x[:, :h], x[:, h:])
    return x


def _body(x0_ref, *refs):
    wrefs = refs[:40]
    orefs = refs[40:]
    groups = []
    idx = 0
    for _ in range(5):
        g = []
        for _ in range(4):
            g.append((wrefs[idx][...], wrefs[idx + 1][...]))
            idx += 2
        groups.append(g)
    dyn_pre, dyn_post, branch_proj, conv_mlp, pp_conv_mlp = groups
    E = _E

    def lvl(x, L):
        return x[:, _off(L) * E:_off(L + 1) * E]

    def hlvs(xall, top):
        h = _mlp(dyn_pre, xall)
        ssum = None
        smax = None
        for L in range(top + 1):
            seg = lvl(h, L)
            s = _fold_sum(seg, L)
            m = _fold_max(seg, L)
            ssum = s if ssum is None else ssum + s
            smax = m if smax is None else jnp.maximum(smax, m)
        cnt = float(2 ** (top + 1) - 1)
        W1, b1 = dyn_post[0]
        h = _leaky(jnp.dot(W1[:, :_NF], ssum / cnt,
                           preferred_element_type=jnp.float32)
                   + jnp.dot(W1[:, _NF:], smax,
                             preferred_element_type=jnp.float32) + b1)
        return _mlp(dyn_post[1:], h)

    def gin(xall, gf, params, top):
        W1, b1 = params[0]
        parts = [xall[:, :E]]
        for L in range(1, top + 1):
            prev = lvl(xall, L - 1)
            parts.append(lvl(xall, L) + jnp.concatenate([prev, prev], axis=1))
        xsum = jnp.concatenate(parts, axis=1)
        gterm = jnp.dot(W1[:, _NF:], gf, preferred_element_type=jnp.float32)
        gparts = [gterm + b1]
        t = 2.0 * gterm + b1
        for L in range(1, top + 1):
            t = jnp.concatenate([t, t], axis=1)
            gparts.append(t)
        h = _leaky(jnp.dot(W1[:, :_NF], xsum,
                           preferred_element_type=jnp.float32)
                   + jnp.concatenate(gparts, axis=1))
        return _mlp(params[1:], h)

    xall = x0_ref[...]
    for s in range(_NL - 1):
        gf = hlvs(xall, s)
        W1, b1 = branch_proj[0]
        gt = jnp.dot(W1[:, _NF:], gf, preferred_element_type=jnp.float32) + b1
        for _ in range(s):
            gt = jnp.concatenate([gt, gt], axis=1)
        h = _leaky(jnp.dot(W1[:, :_NF], lvl(xall, s),
                           preferred_element_type=jnp.float32) + gt)
        ch = _mlp(branch_proj[1:], h)
        child = jnp.concatenate([ch[:_NF, :], ch[_NF:, :]], axis=1)
        xall = jnp.concatenate([xall, child], axis=1)
        xall = gin(xall, gf, conv_mlp, s + 1)
    gf = hlvs(xall, _NL - 1)
    xall = gin(xall, gf, pp_conv_mlp, _NL - 1)
    gf = hlvs(xall, _NL - 1)
    Wl, bl = pp_conv_mlp[3]
    pp_last3 = pp_conv_mlp[:3] + [(Wl[:3, :], bl[:3, :])]
    y = gin(xall, gf, pp_last3, _NL - 1)
    for L in range(_NL):
        yl = lvl(y, L)
        blocks = [yl[:, _bitrev(j, L) * E:(_bitrev(j, L) + 1) * E]
                  for j in range(2 ** L)]
        orefs[L][...] = blocks[0] if L == 0 else jnp.concatenate(blocks, axis=1)


def kernel(random_vector, dyn_pre, dyn_post, branch_proj, conv_mlp, pp_conv_mlp):
    x0 = random_vector.reshape(_NE, _NF).T
    wflat = []
    for g in (dyn_pre, dyn_post, branch_proj, conv_mlp, pp_conv_mlp):
        for W, b in g:
            wflat.append(W.T)
            wflat.append(b.reshape(-1, 1))
    nblk = _NE // _E
    in_specs = [pl.BlockSpec((_NF, _E), lambda b: (0, b))]
    for w in wflat:
        in_specs.append(pl.BlockSpec(w.shape, lambda b: (0, 0)))
    out_shapes = [jax.ShapeDtypeStruct((3, _NE * 2 ** L), jnp.float32)
                  for L in range(_NL)]
    out_specs = [pl.BlockSpec((3, _E * 2 ** L), lambda b: (0, b))
                 for L in range(_NL)]
    outs = pl.pallas_call(
        _body,
        grid=(nblk,),
        in_specs=in_specs,
        out_specs=out_specs,
        out_shape=out_shapes,
        compiler_params=pltpu.CompilerParams(
            dimension_semantics=("parallel",)),
    )(x0, *wflat)
    res = []
    for L, o in enumerate(outs):
        o4 = o.reshape(3, nblk, 2 ** L, _E)
        res.append(o4.transpose(1, 3, 2, 0).reshape(_NE * 2 ** L, 3))
    return jnp.concatenate(res, axis=0)


# speedup vs baseline: 1.1327x; 1.1327x over previous
import jax
import jax.numpy as jnp
from jax.experimental import pallas as pl
from jax.experimental.pallas import tpu as pltpu

_NE = 1024
_NL = 8
_NF = 32
_K = 4
_E2 = 128


def _off(level):
    return 2 ** level - 1


def _bitrev(j, bits):
    r = 0
    for _ in range(bits):
        r = (r << 1) | (j & 1)
        j >>= 1
    return r


def _leaky(x):
    return jnp.maximum(x, 0.1 * x)


def _mlp(params, x):
    n = len(params)
    for i, (Wt, b) in enumerate(params):
        x = jnp.dot(Wt, x, preferred_element_type=jnp.float32) + b
        if i < n - 1:
            x = _leaky(x)
    return x


def _fold_sum(x, steps):
    for _ in range(steps):
        h = x.shape[1] // 2
        x = x[:, :h] + x[:, h:]
    return x


def _fold_max(x, steps):
    for _ in range(steps):
        h = x.shape[1] // 2
        x = jnp.maximum(x[:, :h], x[:, h:])
    return x


def _unpack(wrefs):
    groups = []
    idx = 0
    for _ in range(5):
        g = []
        for _ in range(4):
            g.append((wrefs[idx][...], wrefs[idx + 1][...]))
            idx += 2
        groups.append(g)
    return groups


def _lvl(x, L, E):
    return x[:, _off(L) * E:_off(L + 1) * E]


def _hlvs(dyn_pre, dyn_post, xall, top, E):
    h = _mlp(dyn_pre, xall)
    ssum = None
    smax = None
    for L in range(top + 1):
        seg = _lvl(h, L, E)
        s = _fold_sum(seg, L)
        m = _fold_max(seg, L)
        ssum = s if ssum is None else ssum + s
        smax = m if smax is None else jnp.maximum(smax, m)
    cnt = float(2 ** (top + 1) - 1)
    W1, b1 = dyn_post[0]
    h = _leaky(jnp.dot(W1[:, :_NF], ssum / cnt,
                       preferred_element_type=jnp.float32)
               + jnp.dot(W1[:, _NF:], smax,
                         preferred_element_type=jnp.float32) + b1)
    return _mlp(dyn_post[1:], h)


def _gin(params, xall, gf, top, E):
    W1, b1 = params[0]
    parts = [xall[:, :E]]
    for L in range(1, top + 1):
        prev = _lvl(xall, L - 1, E)
        parts.append(_lvl(xall, L, E) + jnp.concatenate([prev, prev], axis=1))
    xsum = jnp.concatenate(parts, axis=1)
    gterm = jnp.dot(W1[:, _NF:], gf, preferred_element_type=jnp.float32)
    gparts = [gterm + b1]
    t = 2.0 * gterm + b1
    for L in range(1, top + 1):
        t = jnp.concatenate([t, t], axis=1)
        gparts.append(t)
    h = _leaky(jnp.dot(W1[:, :_NF], xsum, preferred_element_type=jnp.float32)
               + jnp.concatenate(gparts, axis=1))
    return _mlp(params[1:], h)


def _split(groups, xall, s, E):
    dyn_pre, dyn_post, branch_proj, conv_mlp, _ = groups
    gf = _hlvs(dyn_pre, dyn_post, xall, s, E)
    W1, b1 = branch_proj[0]
    gt = jnp.dot(W1[:, _NF:], gf, preferred_element_type=jnp.float32) + b1
    for _ in range(s):
        gt = jnp.concatenate([gt, gt], axis=1)
    h = _leaky(jnp.dot(W1[:, :_NF], _lvl(xall, s, E),
                       preferred_element_type=jnp.float32) + gt)
    ch = _mlp(branch_proj[1:], h)
    child = jnp.concatenate([ch[:_NF, :], ch[_NF:, :]], axis=1)
    xall = jnp.concatenate([xall, child], axis=1)
    return _gin(conv_mlp, xall, gf, s + 1, E)


def _p1_body(x0_ref, *refs):
    groups = _unpack(refs[:40])
    orefs = refs[40:]
    E = _NE
    xall = x0_ref[...]
    for s in range(_K):
        xall = _split(groups, xall, s, E)
    for L in range(_K + 1):
        lv = _lvl(xall, L, E)
        if L == 0:
            orefs[L][...] = lv
        else:
            orefs[L][...] = jnp.concatenate(
                [lv[:, bi * E:(bi + 1) * E] for bi in range(2 ** L)], axis=0)


def _p2_body(*refs):
    lrefs = refs[:_K + 1]
    groups = _unpack(refs[_K + 1:_K + 41])
    orefs = refs[_K + 41:]
    E = _E2
    parts = []
    for L in range(_K + 1):
        a = lrefs[L][...]
        parts.extend(a[bi * _NF:(bi + 1) * _NF, :] for bi in range(2 ** L))
    xall = jnp.concatenate(parts, axis=1)
    for s in range(_K, _NL - 1):
        xall = _split(groups, xall, s, E)
    dyn_pre, dyn_post, _, _, pp_conv_mlp = groups
    for _ in range(2):
        gf = _hlvs(dyn_pre, dyn_post, xall, _NL - 1, E)
        xall = _gin(pp_conv_mlp, xall, gf, _NL - 1, E)
    y = xall[:3, :]
    for L in range(_NL):
        yl = _lvl(y, L, E)
        blocks = [yl[:, _bitrev(j, L) * E:(_bitrev(j, L) + 1) * E]
                  for j in range(2 ** L)]
        orefs[L][...] = blocks[0] if L == 0 else jnp.concatenate(blocks, axis=1)


def kernel(random_vector, dyn_pre, dyn_post, branch_proj, conv_mlp, pp_conv_mlp):
    x0 = random_vector.reshape(_NE, _NF).T
    wflat = []
    for g in (dyn_pre, dyn_post, branch_proj, conv_mlp, pp_conv_mlp):
        for W, b in g:
            wflat.append(W.T)
            wflat.append(b.reshape(-1, 1))
    wspecs = [pl.BlockSpec(w.shape, lambda b: (0, 0)) for w in wflat]

    p1_out_shapes = [jax.ShapeDtypeStruct((2 ** L * _NF, _NE), jnp.float32)
                     for L in range(_K + 1)]
    p1_out_specs = [pl.BlockSpec((2 ** L * _NF, _NE), lambda b: (0, 0))
                    for L in range(_K + 1)]
    levels = pl.pallas_call(
        _p1_body,
        grid=(1,),
        in_specs=[pl.BlockSpec((_NF, _NE), lambda b: (0, 0))] + wspecs,
        out_specs=p1_out_specs,
        out_shape=p1_out_shapes,
        compiler_params=pltpu.CompilerParams(
            dimension_semantics=("parallel",)),
    )(x0, *wflat)

    nblk = _NE // _E2
    p2_in_specs = [pl.BlockSpec((2 ** L * _NF, _E2), lambda b: (0, b))
                   for L in range(_K + 1)] + wspecs
    out_shapes = [jax.ShapeDtypeStruct((3, _NE * 2 ** L), jnp.float32)
                  for L in range(_NL)]
    out_specs = [pl.BlockSpec((3, _E2 * 2 ** L), lambda b: (0, b))
                 for L in range(_NL)]
    outs = pl.pallas_call(
        _p2_body,
        grid=(nblk,),
        in_specs=p2_in_specs,
        out_specs=out_specs,
        out_shape=out_shapes,
        compiler_params=pltpu.CompilerParams(
            dimension_semantics=("parallel",)),
    )(*levels, *wflat)

    res = []
    for L, o in enumerate(outs):
        o4 = o.reshape(3, nblk, 2 ** L, _E2)
        res.append(o4.transpose(1, 3, 2, 0).reshape(_NE * 2 ** L, 3))
    return jnp.concatenate(res, axis=0)


# speedup vs baseline: 1.1568x; 1.0213x over previous
import jax
import jax.numpy as jnp
from jax.experimental import pallas as pl
from jax.experimental.pallas import tpu as pltpu

_NE = 1024
_NL = 8
_NF = 32
_K = 5
_E2 = 128


def _off(level):
    return 2 ** level - 1


def _bitrev(j, bits):
    r = 0
    for _ in range(bits):
        r = (r << 1) | (j & 1)
        j >>= 1
    return r


def _leaky(x):
    return jnp.maximum(x, 0.1 * x)


def _mlp(params, x):
    n = len(params)
    for i, (Wt, b) in enumerate(params):
        x = jnp.dot(Wt, x, preferred_element_type=jnp.float32) + b
        if i < n - 1:
            x = _leaky(x)
    return x


def _fold_sum(x, steps):
    for _ in range(steps):
        h = x.shape[1] // 2
        x = x[:, :h] + x[:, h:]
    return x


def _fold_max(x, steps):
    for _ in range(steps):
        h = x.shape[1] // 2
        x = jnp.maximum(x[:, :h], x[:, h:])
    return x


def _unpack(wrefs):
    groups = []
    idx = 0
    for _ in range(5):
        g = []
        for _ in range(4):
            g.append((wrefs[idx][...], wrefs[idx + 1][...]))
            idx += 2
        groups.append(g)
    return groups


def _lvl(x, L, E):
    return x[:, _off(L) * E:_off(L + 1) * E]


def _hlvs(dyn_pre, dyn_post, xall, top, E):
    h = _mlp(dyn_pre, xall)
    ssum = None
    smax = None
    for L in range(top + 1):
        seg = _lvl(h, L, E)
        s = _fold_sum(seg, L)
        m = _fold_max(seg, L)
        ssum = s if ssum is None else ssum + s
        smax = m if smax is None else jnp.maximum(smax, m)
    cnt = float(2 ** (top + 1) - 1)
    W1, b1 = dyn_post[0]
    h = _leaky(jnp.dot(W1[:, :_NF], ssum / cnt,
                       preferred_element_type=jnp.float32)
               + jnp.dot(W1[:, _NF:], smax,
                         preferred_element_type=jnp.float32) + b1)
    return _mlp(dyn_post[1:], h)


def _gin(params, xall, gf, top, E):
    W1, b1 = params[0]
    parts = [xall[:, :E]]
    for L in range(1, top + 1):
        prev = _lvl(xall, L - 1, E)
        parts.append(_lvl(xall, L, E) + jnp.concatenate([prev, prev], axis=1))
    xsum = jnp.concatenate(parts, axis=1)
    gterm = jnp.dot(W1[:, _NF:], gf, preferred_element_type=jnp.float32)
    gparts = [gterm + b1]
    t = 2.0 * gterm + b1
    for L in range(1, top + 1):
        t = jnp.concatenate([t, t], axis=1)
        gparts.append(t)
    h = _leaky(jnp.dot(W1[:, :_NF], xsum, preferred_element_type=jnp.float32)
               + jnp.concatenate(gparts, axis=1))
    return _mlp(params[1:], h)


def _split(groups, xall, s, E):
    dyn_pre, dyn_post, branch_proj, conv_mlp, _ = groups
    gf = _hlvs(dyn_pre, dyn_post, xall, s, E)
    W1, b1 = branch_proj[0]
    gt = jnp.dot(W1[:, _NF:], gf, preferred_element_type=jnp.float32) + b1
    for _ in range(s):
        gt = jnp.concatenate([gt, gt], axis=1)
    h = _leaky(jnp.dot(W1[:, :_NF], _lvl(xall, s, E),
                       preferred_element_type=jnp.float32) + gt)
    ch = _mlp(branch_proj[1:], h)
    child = jnp.concatenate([ch[:_NF, :], ch[_NF:, :]], axis=1)
    xall = jnp.concatenate([xall, child], axis=1)
    return _gin(conv_mlp, xall, gf, s + 1, E)


def _p1_body(x0_ref, *refs):
    groups = _unpack(refs[:40])
    orefs = refs[40:]
    E = _NE
    xall = x0_ref[...]
    for s in range(_K):
        xall = _split(groups, xall, s, E)
    for L in range(_K + 1):
        lv = _lvl(xall, L, E)
        if L == 0:
            orefs[L][...] = lv
        else:
            orefs[L][...] = jnp.concatenate(
                [lv[:, bi * E:(bi + 1) * E] for bi in range(2 ** L)], axis=0)


def _p2_body(*refs):
    lrefs = refs[:_K + 1]
    groups = _unpack(refs[_K + 1:_K + 41])
    orefs = refs[_K + 41:]
    E = _E2
    parts = []
    for L in range(_K + 1):
        a = lrefs[L][...]
        parts.extend(a[bi * _NF:(bi + 1) * _NF, :] for bi in range(2 ** L))
    xall = jnp.concatenate(parts, axis=1)
    for s in range(_K, _NL - 1):
        xall = _split(groups, xall, s, E)
    dyn_pre, dyn_post, _, _, pp_conv_mlp = groups
    for _ in range(2):
        gf = _hlvs(dyn_pre, dyn_post, xall, _NL - 1, E)
        xall = _gin(pp_conv_mlp, xall, gf, _NL - 1, E)
    y = xall[:3, :]
    for L in range(_NL):
        yl = _lvl(y, L, E)
        blocks = [yl[:, _bitrev(j, L) * E:(_bitrev(j, L) + 1) * E]
                  for j in range(2 ** L)]
        orefs[L][...] = blocks[0] if L == 0 else jnp.concatenate(blocks, axis=1)


def kernel(random_vector, dyn_pre, dyn_post, branch_proj, conv_mlp, pp_conv_mlp):
    x0 = random_vector.reshape(_NE, _NF).T
    wflat = []
    for g in (dyn_pre, dyn_post, branch_proj, conv_mlp, pp_conv_mlp):
        for W, b in g:
            wflat.append(W.T)
            wflat.append(b.reshape(-1, 1))
    wspecs = [pl.BlockSpec(w.shape, lambda b: (0, 0)) for w in wflat]

    p1_out_shapes = [jax.ShapeDtypeStruct((2 ** L * _NF, _NE), jnp.float32)
                     for L in range(_K + 1)]
    p1_out_specs = [pl.BlockSpec((2 ** L * _NF, _NE), lambda b: (0, 0))
                    for L in range(_K + 1)]
    levels = pl.pallas_call(
        _p1_body,
        grid=(1,),
        in_specs=[pl.BlockSpec((_NF, _NE), lambda b: (0, 0))] + wspecs,
        out_specs=p1_out_specs,
        out_shape=p1_out_shapes,
        compiler_params=pltpu.CompilerParams(
            dimension_semantics=("parallel",)),
    )(x0, *wflat)

    nblk = _NE // _E2
    p2_in_specs = [pl.BlockSpec((2 ** L * _NF, _E2), lambda b: (0, b))
                   for L in range(_K + 1)] + wspecs
    out_shapes = [jax.ShapeDtypeStruct((3, _NE * 2 ** L), jnp.float32)
                  for L in range(_NL)]
    out_specs = [pl.BlockSpec((3, _E2 * 2 ** L), lambda b: (0, b))
                 for L in range(_NL)]
    outs = pl.pallas_call(
        _p2_body,
        grid=(nblk,),
        in_specs=p2_in_specs,
        out_specs=out_specs,
        out_shape=out_shapes,
        compiler_params=pltpu.CompilerParams(
            dimension_semantics=("parallel",)),
    )(*levels, *wflat)

    res = []
    for L, o in enumerate(outs):
        o4 = o.reshape(3, nblk, 2 ** L, _E2)
        res.append(o4.transpose(1, 3, 2, 0).reshape(_NE * 2 ** L, 3))
    return jnp.concatenate(res, axis=0)
